# Initial kernel scaffold; baseline (speedup 1.0000x reference)
import jax
import jax.numpy as jnp
from jax.experimental import pallas as pl


def kernel(logits, labels, edge_weights, left_labels, right_labels):
    raise NotImplementedError("write your pallas kernel here")


# speedup vs baseline: 1.4589x; 1.4589x over previous
import functools

import jax
import jax.numpy as jnp
from jax import lax
from jax.experimental import pallas as pl
from jax.experimental.pallas import tpu as pltpu
from jax.experimental.pallas import tpu_sc as plsc

_LANES = 16


def _gram_body(logits_ref, labels_ref, out_ref):
    lab = labels_ref[...]
    lgt = logits_ref[...]
    ann = (lab == 0.0) | (lab == 1.0)
    y = jnp.where(ann, 2.0 * lab - 1.0, 2.0 * jax.nn.sigmoid(lgt) - 1.0)
    g = lax.dot_general(y, y, (((0,), (0,)), ((), ())),
                        preferred_element_type=jnp.float32)

    @pl.when(pl.program_id(0) == 0)
    def _():
        out_ref[...] = jnp.zeros_like(out_ref)

    out_ref[...] += g


@functools.lru_cache(maxsize=None)
def _make_gram(batch, n, block):
    grid = batch // block
    return pl.pallas_call(
        _gram_body,
        grid=(grid,),
        in_specs=[
            pl.BlockSpec((block, n), lambda i: (i, 0)),
            pl.BlockSpec((block, n), lambda i: (i, 0)),
        ],
        out_specs=pl.BlockSpec((n, n), lambda i: (0, 0)),
        out_shape=jax.ShapeDtypeStruct((n, n), jnp.float32),
    )


@functools.lru_cache(maxsize=None)
def _make_edge_reduce(n, e_pad, scale):
    mesh = plsc.VectorSubcoreMesh(core_axis_name="c", subcore_axis_name="s")
    chunks = e_pad // _LANES

    @functools.partial(
        pl.kernel,
        mesh=mesh,
        out_type=jax.ShapeDtypeStruct((_LANES,), jnp.float32),
        compiler_params=pltpu.CompilerParams(needs_layout_passes=False),
        scratch_types=[
            pltpu.VMEM((n * n,), jnp.float32),
            pltpu.VMEM((e_pad,), jnp.int32),
            pltpu.VMEM((e_pad,), jnp.int32),
            pltpu.VMEM((e_pad,), jnp.float32),
            pltpu.VMEM((_LANES,), jnp.float32),
        ],
    )
    def k(g_hbm, l_hbm, r_hbm, w_hbm, out_hbm, g_v, l_v, r_v, w_v, o_v):
        wid = lax.axis_index("s") * 2 + lax.axis_index("c")

        @pl.when(wid == 0)
        def _():
            pltpu.sync_copy(g_hbm, g_v)
            pltpu.sync_copy(l_hbm, l_v)
            pltpu.sync_copy(r_hbm, r_v)
            pltpu.sync_copy(w_hbm, w_v)
            acc = jnp.zeros((_LANES,), jnp.float32)
            for c in range(chunks):
                sl = pl.ds(c * _LANES, _LANES)
                li = l_v[sl]
                ri = r_v[sl]
                we = w_v[sl]
                gll = plsc.load_gather(g_v, [li * (n + 1)])
                grr = plsc.load_gather(g_v, [ri * (n + 1)])
                glr = plsc.load_gather(g_v, [li * n + ri])
                acc = acc + we * (gll + grr - 2.0 * glr)
            total = jnp.sum(acc) * scale
            o_v[...] = jnp.full((_LANES,), total, jnp.float32)
            pltpu.sync_copy(o_v, out_hbm)

    return k


def kernel(logits, labels, edge_weights, left_labels, right_labels):
    batch, n = logits.shape
    e = left_labels.shape[0]
    e_pad = ((e + _LANES - 1) // _LANES) * _LANES
    pad = e_pad - e
    g = _make_gram(batch, n, 2048)(logits, labels)
    l_p = jnp.pad(left_labels, (0, pad))
    r_p = jnp.pad(right_labels, (0, pad))
    w_p = jnp.pad(edge_weights, (0, pad))
    scale = 0.1 / (batch * e)
    out = _make_edge_reduce(n, e_pad, scale)(g.reshape(n * n), l_p, r_p, w_p)
    return out[0]
